# Initial kernel scaffold; baseline (speedup 1.0000x reference)
import jax
import jax.numpy as jnp
from jax.experimental import pallas as pl


def kernel(x, pos, batch, W1, b1, g1, be1, W2, b2, g2, be2, W3, b3, g3, be3, Wc1, bc1, gc1, bec1, Wc2, bc2, gc2, bec2):
    raise NotImplementedError("write your pallas kernel here")


# speedup vs baseline: 8.0132x; 8.0132x over previous
import jax
import jax.numpy as jnp
from jax.experimental import pallas as pl
from jax.experimental.pallas import tpu as pltpu

N = 10000
NP = 10240
K = 4
NUM_GRAPHS = 2
INF = float("inf")


def _knn_body(fT_ref, q_ref, br_ref, bc_ref, idx_ref):
    i = pl.program_id(0)
    Q, D = q_ref.shape
    C = fT_ref.shape[1]
    q = q_ref[...]
    fT = fT_ref[...]
    row_sq = jnp.sum(q * q, axis=1, keepdims=True)
    col_sq = jnp.sum(fT * fT, axis=0, keepdims=True)
    d = row_sq + col_sq - 2.0 * jnp.dot(q, fT, preferred_element_type=jnp.float32)
    cols = jax.lax.broadcasted_iota(jnp.int32, (Q, C), 1)
    row_g = i * Q + jax.lax.broadcasted_iota(jnp.int32, (Q, C), 0)
    mask = (br_ref[...] != bc_ref[...]) | (cols == row_g)
    d = jnp.where(mask, INF, d)
    outs = []
    for _ in range(K):
        m = jnp.min(d, axis=1, keepdims=True)
        cand = jnp.where(d == m, cols, jnp.int32(2 ** 30))
        j = jnp.min(cand, axis=1, keepdims=True)
        outs.append(j)
        d = jnp.where(cols == j, INF, d)
    idx_ref[...] = jnp.concatenate(outs, axis=1)


def _knn(f, br, bc, q_block=128):
    npad, d_feat = f.shape
    fT = f.T
    return pl.pallas_call(
        _knn_body,
        grid=(npad // q_block,),
        in_specs=[
            pl.BlockSpec((d_feat, npad), lambda i: (0, 0)),
            pl.BlockSpec((q_block, d_feat), lambda i: (i, 0)),
            pl.BlockSpec((q_block, 1), lambda i: (i, 0)),
            pl.BlockSpec((1, npad), lambda i: (0, 0)),
        ],
        out_specs=pl.BlockSpec((q_block, K), lambda i: (i, 0)),
        out_shape=jax.ShapeDtypeStruct((npad, K), jnp.int32),
    )(fT, f, br, bc)


def _mm_body(x_ref, w_ref, b_ref, o_ref):
    o_ref[...] = (
        jnp.dot(x_ref[...], w_ref[...], preferred_element_type=jnp.float32)
        + b_ref[...]
    )


def _matmul(x, w, b, m_block=1024):
    npad, d_in = x.shape
    d_out = w.shape[1]
    return pl.pallas_call(
        _mm_body,
        grid=(npad // m_block,),
        in_specs=[
            pl.BlockSpec((m_block, d_in), lambda i: (i, 0)),
            pl.BlockSpec((d_in, d_out), lambda i: (0, 0)),
            pl.BlockSpec((1, d_out), lambda i: (0, 0)),
        ],
        out_specs=pl.BlockSpec((m_block, d_out), lambda i: (i, 0)),
        out_shape=jax.ShapeDtypeStruct((npad, d_out), jnp.float32),
    )(x, w, b)


def _gather_body(idx_ref, a_ref, b_ref, s_ref, st_ref):
    blk = pl.program_id(0)
    bn, f_dim = a_ref.shape

    @pl.when(blk == 0)
    def _():
        st_ref[...] = jnp.zeros_like(st_ref)

    def body(r, carry):
        se, sq = carry
        a = a_ref[pl.ds(r, 1), :]
        acc = jnp.zeros((1, f_dim), jnp.float32)
        asq = jnp.zeros((1, f_dim), jnp.float32)
        for k in range(K):
            j = idx_ref[r, k]
            bj = b_ref[pl.ds(j, 1), :]
            e = jnp.maximum(a + bj, 0.0)
            acc = acc + e
            asq = asq + e * e
        s_ref[pl.ds(r, 1), :] = acc
        vf = jnp.where(blk * bn + r < N, 1.0, 0.0)
        return (se + vf * acc, sq + vf * asq)

    z = jnp.zeros((1, f_dim), jnp.float32)
    se, sq = jax.lax.fori_loop(0, bn, body, (z, z))
    st_ref[0:1, :] += se
    st_ref[1:2, :] += sq


def _gather_accum(idx, ab, f_dim, n_block=256):
    npad = ab.shape[0]
    return pl.pallas_call(
        _gather_body,
        grid=(npad // n_block,),
        in_specs=[
            pl.BlockSpec((n_block, K), lambda i: (i, 0), memory_space=pltpu.SMEM),
            pl.BlockSpec((n_block, f_dim), lambda i: (i, 0)),
            pl.BlockSpec((npad, f_dim), lambda i: (0, 1)),
        ],
        out_specs=[
            pl.BlockSpec((n_block, f_dim), lambda i: (i, 0)),
            pl.BlockSpec((8, f_dim), lambda i: (0, 0)),
        ],
        out_shape=[
            jax.ShapeDtypeStruct((npad, f_dim), jnp.float32),
            jax.ShapeDtypeStruct((8, f_dim), jnp.float32),
        ],
    )(idx, ab, ab)


def _affine_body(s_ref, al_ref, be_ref, o_ref):
    o_ref[...] = s_ref[...] * al_ref[...] + be_ref[...]


def _affine(s, al, be, m_block=1024):
    npad, f_dim = s.shape
    return pl.pallas_call(
        _affine_body,
        grid=(npad // m_block,),
        in_specs=[
            pl.BlockSpec((m_block, f_dim), lambda i: (i, 0)),
            pl.BlockSpec((1, f_dim), lambda i: (0, 0)),
            pl.BlockSpec((1, f_dim), lambda i: (0, 0)),
        ],
        out_specs=pl.BlockSpec((m_block, f_dim), lambda i: (i, 0)),
        out_shape=jax.ShapeDtypeStruct((npad, f_dim), jnp.float32),
    )(s, al, be)


def _pool_cls_body(h_ref, bt_ref, wc1_ref, bc1_ref, gc1_ref, bec1_ref,
                   wc2_ref, bc2_ref, gc2_ref, bec2_ref, o_ref, sums, cnts):
    i = pl.program_id(0)

    @pl.when(i == 0)
    def _():
        sums[...] = jnp.zeros_like(sums)
        cnts[...] = jnp.zeros_like(cnts)

    h = h_ref[...]
    bt = bt_ref[...]
    for g in range(NUM_GRAPHS):
        mk = jnp.where(bt == float(g), 1.0, 0.0)
        sums[g:g + 1, :] += jnp.sum(h * mk, axis=0, keepdims=True)
        cnts[g:g + 1, 0:1] += jnp.sum(mk, axis=0, keepdims=True)

    @pl.when(i == pl.num_programs(0) - 1)
    def _():
        cnt = jnp.maximum(cnts[:, 0:1], 1.0)
        pooled = sums[...] / cnt
        z = jnp.maximum(
            jnp.dot(pooled, wc1_ref[...], preferred_element_type=jnp.float32)
            + bc1_ref[...], 0.0)
        m = (z[0:1] + z[1:2]) * 0.5
        v = ((z[0:1] - m) ** 2 + (z[1:2] - m) ** 2) * 0.5
        zn = (z - m) / jnp.sqrt(v + 1e-5) * gc1_ref[...] + bec1_ref[...]
        z2 = jnp.maximum(
            jnp.dot(zn, wc2_ref[...], preferred_element_type=jnp.float32)
            + bc2_ref[...], 0.0)
        m2 = (z2[0:1] + z2[1:2]) * 0.5
        v2 = ((z2[0:1] - m2) ** 2 + (z2[1:2] - m2) ** 2) * 0.5
        z2n = (z2 - m2) / jnp.sqrt(v2 + 1e-5) * gc2_ref[...] + bec2_ref[...]
        o_ref[...] = jax.nn.sigmoid(jnp.broadcast_to(z2n, o_ref.shape))


def _pool_cls(h, br, wc1, bc1, gc1, bec1, wc2, bc2, gc2, bec2, m_block=512):
    npad, f_dim = h.shape
    return pl.pallas_call(
        _pool_cls_body,
        grid=(npad // m_block,),
        in_specs=[
            pl.BlockSpec((m_block, f_dim), lambda i: (i, 0)),
            pl.BlockSpec((m_block, 1), lambda i: (i, 0)),
            pl.BlockSpec((f_dim, f_dim), lambda i: (0, 0)),
            pl.BlockSpec((1, f_dim), lambda i: (0, 0)),
            pl.BlockSpec((1, f_dim), lambda i: (0, 0)),
            pl.BlockSpec((1, f_dim), lambda i: (0, 0)),
            pl.BlockSpec((f_dim, 1), lambda i: (0, 0)),
            pl.BlockSpec((1, 1), lambda i: (0, 0)),
            pl.BlockSpec((1, 1), lambda i: (0, 0)),
            pl.BlockSpec((1, 1), lambda i: (0, 0)),
        ],
        out_specs=pl.BlockSpec((8, 128), lambda i: (0, 0)),
        out_shape=jax.ShapeDtypeStruct((8, 128), jnp.float32),
        scratch_shapes=[
            pltpu.VMEM((8, f_dim), jnp.float32),
            pltpu.VMEM((8, 128), jnp.float32),
        ],
    )(h, br, wc1, bc1.reshape(1, -1), gc1.reshape(1, -1), bec1.reshape(1, -1),
      wc2, bc2.reshape(1, 1), gc2.reshape(1, 1), bec2.reshape(1, 1))


def _edge_conv(x_p, idx, W, b, g, be, f_in, f_out):
    wa = W[:f_in]
    wb = W[f_in:2 * f_in]
    wcat = jnp.concatenate([wa - wb, wb], axis=1)
    pad_rows = x_p.shape[1] - f_in
    if pad_rows:
        wcat = jnp.pad(wcat, ((0, pad_rows), (0, 0)))
    bias = jnp.concatenate([b, jnp.zeros_like(b)]).reshape(1, -1)
    ab = _matmul(x_p, wcat, bias)
    s, st = _gather_accum(idx, ab, f_out)
    ec = jnp.float32(N * K)
    m = st[0] / ec
    v = st[1] / ec - m * m
    inv = 1.0 / jnp.sqrt(v + 1e-5)
    alpha = (g * inv / K).reshape(1, -1)
    beta = (be - m * g * inv).reshape(1, -1)
    return _affine(s, alpha, beta)


def kernel(x, pos, batch, W1, b1, g1, be1, W2, b2, g2, be2, W3, b3, g3, be3,
           Wc1, bc1, gc1, bec1, Wc2, bc2, gc2, bec2):
    pad = NP - N
    batch_f = jnp.pad(batch.astype(jnp.float32), (0, pad), constant_values=-1.0)
    br = batch_f.reshape(NP, 1)
    bc = batch_f.reshape(1, NP)
    pos_p = jnp.pad(pos, ((0, pad), (0, 5)))
    x_p = jnp.pad(x, ((0, pad), (0, 11)))

    idx1 = _knn(pos_p, br, bc)
    h1 = _edge_conv(x_p, idx1, W1, b1, g1, be1, 53, 128)

    idx2 = _knn(h1, br, bc)
    h2 = _edge_conv(h1, idx2, W2, b2, g2, be2, 128, 256)

    idx3 = _knn(h2, br, bc)
    h3 = _edge_conv(h2, idx3, W3, b3, g3, be3, 256, 512)

    out = _pool_cls(h3, br, Wc1, bc1, gc1, bec1, Wc2, bc2, gc2, bec2)
    return out[0:2, 0]


# speedup vs baseline: 8.6032x; 1.0736x over previous
import functools

import jax
import jax.numpy as jnp
from jax import lax
from jax.experimental import pallas as pl
from jax.experimental.pallas import tpu as pltpu
from jax.experimental.pallas import tpu_sc as plsc

N = 10000
NP = 10240
K = 4
NUM_GRAPHS = 2
INF = float("inf")


def _knn_body(fT_ref, q_ref, br_ref, bc_ref, idx_ref):
    i = pl.program_id(0)
    Q, D = q_ref.shape
    C = fT_ref.shape[1]
    q = q_ref[...]
    fT = fT_ref[...]
    row_sq = jnp.sum(q * q, axis=1, keepdims=True)
    col_sq = jnp.sum(fT * fT, axis=0, keepdims=True)
    d = row_sq + col_sq - 2.0 * jnp.dot(q, fT, preferred_element_type=jnp.float32)
    cols = jax.lax.broadcasted_iota(jnp.int32, (Q, C), 1)
    row_g = i * Q + jax.lax.broadcasted_iota(jnp.int32, (Q, C), 0)
    mask = (br_ref[...] != bc_ref[...]) | (cols == row_g)
    d = jnp.where(mask, INF, d)
    outs = []
    for _ in range(K):
        m = jnp.min(d, axis=1, keepdims=True)
        cand = jnp.where(d == m, cols, jnp.int32(2 ** 30))
        j = jnp.min(cand, axis=1, keepdims=True)
        outs.append(j)
        d = jnp.where(cols == j, INF, d)
    idx_ref[...] = jnp.concatenate(outs, axis=1)


def _knn(f, br, bc, q_block=128):
    npad, d_feat = f.shape
    fT = f.T
    return pl.pallas_call(
        _knn_body,
        grid=(npad // q_block,),
        in_specs=[
            pl.BlockSpec((d_feat, npad), lambda i: (0, 0)),
            pl.BlockSpec((q_block, d_feat), lambda i: (i, 0)),
            pl.BlockSpec((q_block, 1), lambda i: (i, 0)),
            pl.BlockSpec((1, npad), lambda i: (0, 0)),
        ],
        out_specs=pl.BlockSpec((q_block, K), lambda i: (i, 0)),
        out_shape=jax.ShapeDtypeStruct((npad, K), jnp.int32),
    )(fT, f, br, bc)


def _mm_body(x_ref, w_ref, b_ref, a_ref, bb_ref):
    f_out = a_ref.shape[1]
    res = (
        jnp.dot(x_ref[...], w_ref[...], preferred_element_type=jnp.float32)
        + b_ref[...]
    )
    a_ref[...] = res[:, :f_out]
    bb_ref[...] = res[:, f_out:]


def _matmul_ab(x, w, b, m_block=1024):
    npad, d_in = x.shape
    d_out = w.shape[1]
    f_out = d_out // 2
    return pl.pallas_call(
        _mm_body,
        grid=(npad // m_block,),
        in_specs=[
            pl.BlockSpec((m_block, d_in), lambda i: (i, 0)),
            pl.BlockSpec((d_in, d_out), lambda i: (0, 0)),
            pl.BlockSpec((1, d_out), lambda i: (0, 0)),
        ],
        out_specs=[
            pl.BlockSpec((m_block, f_out), lambda i: (i, 0)),
            pl.BlockSpec((m_block, f_out), lambda i: (i, 0)),
        ],
        out_shape=[
            jax.ShapeDtypeStruct((npad, f_out), jnp.float32),
            jax.ShapeDtypeStruct((npad, f_out), jnp.float32),
        ],
    )(x, w, b)


_SC_NC = 2
_SC_NS = 16
_SC_NW = _SC_NC * _SC_NS
_NODES_W = NP // _SC_NW
_NODES_C = 32
_EDGES_C = _NODES_C * K
_CHUNKS = _NODES_W // _NODES_C


def _sc_gather_body(f_dim, a_hbm, b_hbm, idx_hbm, s_hbm, st_hbm,
                    idx_v, rows_v, a_v, acc_v, sum_v, sq_v, sem):
    nt = f_dim // 16
    wid = lax.axis_index("s") * _SC_NC + lax.axis_index("c")
    base = wid * _NODES_W

    pltpu.sync_copy(idx_hbm.at[pl.ds(base * K, _NODES_W * K)], idx_v)
    zero = jnp.zeros((16,), jnp.float32)
    for t in range(nt):
        sum_v[pl.ds(t * 16, 16)] = zero
        sq_v[pl.ds(t * 16, 16)] = zero

    def chunk_body(c, _):
        nbase = base + c * _NODES_C
        pltpu.async_copy(
            b_hbm.at[idx_v.at[pl.ds(c * _EDGES_C, _EDGES_C)]], rows_v, sem
        ).wait()
        pltpu.sync_copy(a_hbm.at[pl.ds(nbase, _NODES_C)], a_v)

        def node_body(n, _n):
            for t in range(nt):
                sl = pl.ds(t * 16, 16)
                a16 = a_v[n, sl]
                acc = jnp.zeros((16,), jnp.float32)
                asq = jnp.zeros((16,), jnp.float32)
                for k in range(K):
                    b16 = rows_v[K * n + k, sl]
                    e = jnp.maximum(a16 + b16, 0.0)
                    acc = acc + e
                    asq = asq + e * e
                acc_v[n, sl] = acc

                @pl.when(nbase + n < N)
                def _():
                    plsc.addupdate(sum_v.at[sl], acc)
                    plsc.addupdate(sq_v.at[sl], asq)
            return 0

        lax.fori_loop(0, _NODES_C, node_body, 0)
        pltpu.sync_copy(acc_v, s_hbm.at[pl.ds(nbase, _NODES_C)])
        return 0

    lax.fori_loop(0, _CHUNKS, chunk_body, 0)
    pltpu.sync_copy(sum_v, st_hbm.at[wid])
    pltpu.sync_copy(sq_v, st_hbm.at[_SC_NW + wid])


def _gather_accum(idx, a, b, f_dim):
    mesh = plsc.VectorSubcoreMesh(core_axis_name="c", subcore_axis_name="s")
    kern = functools.partial(_sc_gather_body, f_dim)
    run = functools.partial(
        pl.kernel, mesh=mesh,
        out_type=[
            jax.ShapeDtypeStruct((NP, f_dim), jnp.float32),
            jax.ShapeDtypeStruct((2 * _SC_NW, f_dim), jnp.float32),
        ],
        scratch_types=[
            pltpu.VMEM((_NODES_W * K,), jnp.int32),
            pltpu.VMEM((_EDGES_C, f_dim), jnp.float32),
            pltpu.VMEM((_NODES_C, f_dim), jnp.float32),
            pltpu.VMEM((_NODES_C, f_dim), jnp.float32),
            pltpu.VMEM((f_dim,), jnp.float32),
            pltpu.VMEM((f_dim,), jnp.float32),
            pltpu.SemaphoreType.DMA,
        ],
    )(kern)
    return run(a, b, idx.reshape(-1))


def _affine_body(s_ref, st_ref, g_ref, be_ref, o_ref):
    st = st_ref[...]
    ec = jnp.float32(N * K)
    m = jnp.sum(st[:_SC_NW], axis=0, keepdims=True) / ec
    v = jnp.sum(st[_SC_NW:], axis=0, keepdims=True) / ec - m * m
    inv = 1.0 / jnp.sqrt(v + 1e-5)
    alpha = g_ref[...] * inv / K
    beta = be_ref[...] - m * g_ref[...] * inv
    o_ref[...] = s_ref[...] * alpha + beta


def _affine(s, st, g, be, m_block=1024):
    npad, f_dim = s.shape
    return pl.pallas_call(
        _affine_body,
        grid=(npad // m_block,),
        in_specs=[
            pl.BlockSpec((m_block, f_dim), lambda i: (i, 0)),
            pl.BlockSpec((2 * _SC_NW, f_dim), lambda i: (0, 0)),
            pl.BlockSpec((1, f_dim), lambda i: (0, 0)),
            pl.BlockSpec((1, f_dim), lambda i: (0, 0)),
        ],
        out_specs=pl.BlockSpec((m_block, f_dim), lambda i: (i, 0)),
        out_shape=jax.ShapeDtypeStruct((npad, f_dim), jnp.float32),
    )(s, st, g.reshape(1, -1), be.reshape(1, -1))


def _pool_cls_body(h_ref, bt_ref, wc1_ref, bc1_ref, gc1_ref, bec1_ref,
                   wc2_ref, bc2_ref, gc2_ref, bec2_ref, o_ref, sums, cnts):
    i = pl.program_id(0)

    @pl.when(i == 0)
    def _():
        sums[...] = jnp.zeros_like(sums)
        cnts[...] = jnp.zeros_like(cnts)

    h = h_ref[...]
    bt = bt_ref[...]
    for g in range(NUM_GRAPHS):
        mk = jnp.where(bt == float(g), 1.0, 0.0)
        sums[g:g + 1, :] += jnp.sum(h * mk, axis=0, keepdims=True)
        cnts[g:g + 1, 0:1] += jnp.sum(mk, axis=0, keepdims=True)

    @pl.when(i == pl.num_programs(0) - 1)
    def _():
        cnt = jnp.maximum(cnts[:, 0:1], 1.0)
        pooled = sums[...] / cnt
        z = jnp.maximum(
            jnp.dot(pooled, wc1_ref[...], preferred_element_type=jnp.float32)
            + bc1_ref[...], 0.0)
        m = (z[0:1] + z[1:2]) * 0.5
        v = ((z[0:1] - m) ** 2 + (z[1:2] - m) ** 2) * 0.5
        zn = (z - m) / jnp.sqrt(v + 1e-5) * gc1_ref[...] + bec1_ref[...]
        z2 = jnp.maximum(
            jnp.dot(zn, wc2_ref[...], preferred_element_type=jnp.float32)
            + bc2_ref[...], 0.0)
        m2 = (z2[0:1] + z2[1:2]) * 0.5
        v2 = ((z2[0:1] - m2) ** 2 + (z2[1:2] - m2) ** 2) * 0.5
        z2n = (z2 - m2) / jnp.sqrt(v2 + 1e-5) * gc2_ref[...] + bec2_ref[...]
        o_ref[...] = jax.nn.sigmoid(jnp.broadcast_to(z2n, o_ref.shape))


def _pool_cls(h, br, wc1, bc1, gc1, bec1, wc2, bc2, gc2, bec2, m_block=512):
    npad, f_dim = h.shape
    return pl.pallas_call(
        _pool_cls_body,
        grid=(npad // m_block,),
        in_specs=[
            pl.BlockSpec((m_block, f_dim), lambda i: (i, 0)),
            pl.BlockSpec((m_block, 1), lambda i: (i, 0)),
            pl.BlockSpec((f_dim, f_dim), lambda i: (0, 0)),
            pl.BlockSpec((1, f_dim), lambda i: (0, 0)),
            pl.BlockSpec((1, f_dim), lambda i: (0, 0)),
            pl.BlockSpec((1, f_dim), lambda i: (0, 0)),
            pl.BlockSpec((f_dim, 1), lambda i: (0, 0)),
            pl.BlockSpec((1, 1), lambda i: (0, 0)),
            pl.BlockSpec((1, 1), lambda i: (0, 0)),
            pl.BlockSpec((1, 1), lambda i: (0, 0)),
        ],
        out_specs=pl.BlockSpec((8, 128), lambda i: (0, 0)),
        out_shape=jax.ShapeDtypeStruct((8, 128), jnp.float32),
        scratch_shapes=[
            pltpu.VMEM((8, f_dim), jnp.float32),
            pltpu.VMEM((8, 128), jnp.float32),
        ],
    )(h, br, wc1, bc1.reshape(1, -1), gc1.reshape(1, -1), bec1.reshape(1, -1),
      wc2, bc2.reshape(1, 1), gc2.reshape(1, 1), bec2.reshape(1, 1))


def _edge_conv(x_p, idx, W, b, g, be, f_in, f_out):
    wa = W[:f_in]
    wb = W[f_in:2 * f_in]
    wcat = jnp.concatenate([wa - wb, wb], axis=1)
    pad_rows = x_p.shape[1] - f_in
    if pad_rows:
        wcat = jnp.pad(wcat, ((0, pad_rows), (0, 0)))
    bias = jnp.concatenate([b, jnp.zeros_like(b)]).reshape(1, -1)
    a, bb = _matmul_ab(x_p, wcat, bias)
    s, st = _gather_accum(idx, a, bb, f_out)
    return _affine(s, st, g, be)


def kernel(x, pos, batch, W1, b1, g1, be1, W2, b2, g2, be2, W3, b3, g3, be3,
           Wc1, bc1, gc1, bec1, Wc2, bc2, gc2, bec2):
    pad = NP - N
    batch_f = jnp.pad(batch.astype(jnp.float32), (0, pad), constant_values=-1.0)
    br = batch_f.reshape(NP, 1)
    bc = batch_f.reshape(1, NP)
    pos_p = jnp.pad(pos, ((0, pad), (0, 5)))
    x_p = jnp.pad(x, ((0, pad), (0, 11)))

    idx1 = _knn(pos_p, br, bc)
    h1 = _edge_conv(x_p, idx1, W1, b1, g1, be1, 53, 128)

    idx2 = _knn(h1, br, bc)
    h2 = _edge_conv(h1, idx2, W2, b2, g2, be2, 128, 256)

    idx3 = _knn(h2, br, bc)
    h3 = _edge_conv(h2, idx3, W3, b3, g3, be3, 256, 512)

    out = _pool_cls(h3, br, Wc1, bc1, gc1, bec1, Wc2, bc2, gc2, bec2)
    return out[0:2, 0]
